# baseline (speedup 1.0000x reference)
import functools

import jax
import jax.numpy as jnp
from jax import lax
from jax.experimental import pallas as pl
from jax.experimental.pallas import tpu as pltpu
from jax.experimental.pallas import tpu_sc as plsc

_NC, _NS = 2, 16
_K = 100
_BN_EPS = 1e-5


def _segsum_sc(h, src_idx, dst2d, n, d, split_edges):
    if split_edges:
        gw = d
        h2 = h
        nw = _NC * _NS
    else:
        gw = d // 2
        h2 = h.reshape(2 * n, gw)
        nw = _NS
    nch = dst2d.shape[0]
    cpw = nch // nw
    zr = 200
    nzc = n // zr
    ztrip = (nzc + _NS - 1) // _NS
    mesh = plsc.VectorSubcoreMesh(
        core_axis_name="c", subcore_axis_name="s",
        num_cores=_NC, num_subcores=_NS)

    @functools.partial(
        pl.kernel,
        out_type=jax.ShapeDtypeStruct((_NC * n, gw), jnp.float32),
        mesh=mesh,
        scratch_types=[
            pltpu.VMEM((cpw, _K), jnp.int32),
            pltpu.VMEM((cpw, _K), jnp.int32),
            [pltpu.VMEM((_K, gw), jnp.float32) for _ in range(4)],
            pltpu.VMEM((zr, gw), jnp.float32),
            pltpu.VMEM_SHARED((n, gw), jnp.float32),
            [pltpu.SemaphoreType.DMA for _ in range(4)],
        ],
        compiler_params=pltpu.CompilerParams(use_tc_tiling_on_sc=False),
    )
    def k(h_hbm, src_hbm, dst_hbm, out_hbm, srcb, dstb, bufs, zbuf, acc, sems):
        c = lax.axis_index("c")
        s = lax.axis_index("s")

        zeros16 = jnp.zeros((16,), jnp.float32)

        def zrow(i, _):
            def zcol(j, _):
                zbuf[i, pl.ds(j * 16, 16)] = zeros16
                return 0
            return lax.fori_loop(0, gw // 16, zcol, 0)

        lax.fori_loop(0, zr, zrow, 0)

        def zcopy(t, _):
            ch = jnp.minimum(s + t * _NS, nzc - 1)
            pltpu.sync_copy(zbuf, acc.at[pl.ds(ch * zr, zr)])
            return 0

        lax.fori_loop(0, ztrip, zcopy, 0)

        if split_edges:
            wid = s * _NC + c
            pltpu.sync_copy(src_hbm.at[pl.ds(wid * cpw, cpw)], srcb)
            pltpu.sync_copy(dst_hbm.at[pl.ds(wid * cpw, cpw)], dstb)
        else:
            pltpu.sync_copy(src_hbm.at[pl.ds(c * nch + s * cpw, cpw)], srcb)
            pltpu.sync_copy(dst_hbm.at[pl.ds(s * cpw, cpw)], dstb)
        plsc.subcore_barrier()

        for l in range(4):
            pltpu.async_copy(h_hbm.at[srcb.at[l]], bufs[l], sems[l])

        def body(i, _):
            for l in range(4):
                j = 4 * i + l
                pltpu.make_async_copy(
                    h_hbm.at[srcb.at[j]], bufs[l], sems[l]).wait()
                pltpu.sync_copy(bufs[l], acc.at[dstb.at[j]], add=True)
                jn = jnp.minimum(4 * i + 4 + l, cpw - 4 + l)
                pltpu.async_copy(h_hbm.at[srcb.at[jn]], bufs[l], sems[l])
            return 0

        lax.fori_loop(0, cpw // 4, body, 0)
        for l in range(4):
            pltpu.make_async_copy(
                h_hbm.at[srcb.at[l]], bufs[l], sems[l]).wait()
        plsc.subcore_barrier()

        def wcopy(t, _):
            ch = jnp.minimum(s + t * _NS, nzc - 1)
            pltpu.sync_copy(acc.at[pl.ds(ch * zr, zr)],
                            out_hbm.at[pl.ds(c * n + ch * zr, zr)])
            return 0

        lax.fori_loop(0, ztrip, wcopy, 0)

    return k(h2, src_idx, dst2d)


def _deg_sc(dst2d, n):
    nch = dst2d.shape[0]
    cpw = nch // (_NC * _NS)
    zr = 200
    nzc = n // zr
    ztrip = (nzc + _NS - 1) // _NS
    mesh = plsc.VectorSubcoreMesh(
        core_axis_name="c", subcore_axis_name="s",
        num_cores=_NC, num_subcores=_NS)

    @functools.partial(
        pl.kernel,
        out_type=jax.ShapeDtypeStruct((_NC * n, 16), jnp.float32),
        mesh=mesh,
        scratch_types=[
            pltpu.VMEM((cpw, _K), jnp.int32),
            pltpu.VMEM((_K, 16), jnp.float32),
            pltpu.VMEM((zr, 16), jnp.float32),
            pltpu.VMEM_SHARED((n, 16), jnp.float32),
        ],
        compiler_params=pltpu.CompilerParams(use_tc_tiling_on_sc=False),
    )
    def k(dst_hbm, out_hbm, dstb, ones_b, zbuf, acc):
        c = lax.axis_index("c")
        s = lax.axis_index("s")
        wid = s * _NC + c

        ones16 = jnp.ones((16,), jnp.float32)
        zeros16 = jnp.zeros((16,), jnp.float32)

        def orow(i, _):
            ones_b[i, pl.ds(0, 16)] = ones16
            return 0

        lax.fori_loop(0, _K, orow, 0)

        def zrow(i, _):
            zbuf[i, pl.ds(0, 16)] = zeros16
            return 0

        lax.fori_loop(0, zr, zrow, 0)

        def zcopy(t, _):
            ch = jnp.minimum(s + t * _NS, nzc - 1)
            pltpu.sync_copy(zbuf, acc.at[pl.ds(ch * zr, zr)])
            return 0

        lax.fori_loop(0, ztrip, zcopy, 0)

        pltpu.sync_copy(dst_hbm.at[pl.ds(wid * cpw, cpw)], dstb)
        plsc.subcore_barrier()

        def body(j, _):
            pltpu.sync_copy(ones_b, acc.at[dstb.at[j]], add=True)
            return 0

        lax.fori_loop(0, cpw, body, 0)
        plsc.subcore_barrier()

        def wcopy(t, _):
            ch = jnp.minimum(s + t * _NS, nzc - 1)
            pltpu.sync_copy(acc.at[pl.ds(ch * zr, zr)],
                            out_hbm.at[pl.ds(c * n + ch * zr, zr)])
            return 0

        lax.fori_loop(0, ztrip, wcopy, 0)

    return k(dst2d)


def _bn(u):
    mu = jnp.mean(u, axis=0, keepdims=True)
    var = jnp.mean((u - mu) * (u - mu), axis=0, keepdims=True)
    return (u - mu) / jnp.sqrt(var + _BN_EPS)


def _stem(x, stem_w):
    n = x.shape[0]
    m = stem_w.shape[1]

    def body(x_ref, w_ref, o_ref):
        u = jnp.dot(x_ref[...], w_ref[...], preferred_element_type=jnp.float32)
        o_ref[...] = _bn(u)

    return pl.pallas_call(
        body, out_shape=jax.ShapeDtypeStruct((n, m), jnp.float32),
    )(x, stem_w)


def _pre(s0_list, s1_list, p0, p1):
    n = s0_list[0].shape[0]
    c = p0.shape[1]
    n0, n1 = len(s0_list), len(s1_list)

    def body(*refs):
        s0r = refs[:n0]
        s1r = refs[n0:n0 + n1]
        p0r = refs[n0 + n1]
        p1r = refs[n0 + n1 + 1]
        o_ref = refs[n0 + n1 + 2]
        off = 0
        u0 = jnp.zeros((n, c), jnp.float32)
        for part in s0r:
            w = part.shape[1]
            u0 = u0 + jnp.dot(part[...], p0r[pl.ds(off, w), :],
                              preferred_element_type=jnp.float32)
            off += w
        off = 0
        u1 = jnp.zeros((n, c), jnp.float32)
        for part in s1r:
            w = part.shape[1]
            u1 = u1 + jnp.dot(part[...], p1r[pl.ds(off, w), :],
                              preferred_element_type=jnp.float32)
            off += w
        h0 = _bn(jax.nn.relu(u0))
        h1 = _bn(jax.nn.relu(u1))
        o_ref[...] = jnp.concatenate([h0, h1], axis=1)

    return pl.pallas_call(
        body, out_shape=jax.ShapeDtypeStruct((n, 2 * c), jnp.float32),
    )(*s0_list, *s1_list, p0, p1)


def _step2(h01, mp01, extras, degp, wb, al, out_assign, state_assign,
           base=None):
    c = wb.shape[1] // 3
    k = wb.shape[0]
    if h01 is not None:
        n, d01 = h01.shape
    else:
        n = extras[0][0].shape[0]
        d01 = 0
    r = 2000 if n % 2000 == 0 else n
    g = n // r
    two_out = any(o == 1 for o in out_assign)

    def im_p0(i):
        return (i, 0)

    def im_p1(i):
        return (i + g, 0)

    in_specs = []
    args = []
    if h01 is not None:
        in_specs += [
            pl.BlockSpec((r, d01), im_p0),
            pl.BlockSpec((r, c), im_p0),
            pl.BlockSpec((r, c), im_p1),
        ]
        args += [h01, mp01, mp01]
    for (s_j, mp_j) in extras:
        in_specs += [
            pl.BlockSpec((r, c), im_p0),
            pl.BlockSpec((r, c), im_p0),
            pl.BlockSpec((r, c), im_p1),
        ]
        args += [s_j, mp_j, mp_j]
    if base is not None:
        in_specs += [pl.BlockSpec((r, c), im_p0)]
        args += [base]
    in_specs += [
        pl.BlockSpec((r, 16), im_p0),
        pl.BlockSpec((r, 16), im_p1),
        pl.BlockSpec((k, 3 * c, 4 * c), lambda i: (0, 0, 0)),
        pl.BlockSpec((k, 8), lambda i: (0, 0)),
    ]
    args += [degp, degp, wb, jnp.pad(al, ((0, 0), (0, 2)))]
    nh = 3 if h01 is not None else 0
    nex = len(extras)
    nb = 1 if base is not None else 0
    nfirst = 2 if h01 is not None else 0

    def body(*refs):
        ex = refs[nh:nh + 3 * nex]
        base_ref = refs[nh + 3 * nex] if nb else None
        dg0, dg1, wb_ref, al_ref = refs[nh + 3 * nex + nb:
                                        nh + 3 * nex + nb + 4]
        o_refs = refs[nh + 3 * nex + nb + 4:]

        deg = dg0[:, 0:1] + dg1[:, 0:1]
        rdeg = 1.0 / jnp.maximum(deg, 1.0)
        alv = al_ref[...][:, 0:6]
        w = jax.nn.softmax(alv, axis=-1)
        wg = jnp.reshape(
            jnp.broadcast_to(w[:, 2:6][:, :, None], (k, 4, c)), (k, 4 * c))

        xcats = {}
        hs = {}
        for sid in sorted(set(state_assign)):
            if sid < nfirst:
                h = refs[0][:, sid * c:(sid + 1) * c]
                msum = refs[1][...] if sid == 0 else refs[2][...]
            else:
                e = sid - nfirst
                h = ex[3 * e][...]
                msum = ex[3 * e + 1][...] + ex[3 * e + 2][...]
            hs[sid] = h
            xcats[sid] = jnp.concatenate([h, msum * rdeg, msum], axis=1)

        accs = [base_ref[...] if nb else jnp.zeros((r, c), jnp.float32),
                jnp.zeros((r, c), jnp.float32)]
        for j in range(k):
            sid = state_assign[j]
            y = jax.nn.relu(jnp.dot(xcats[sid], wb_ref[j],
                                    preferred_element_type=jnp.float32))
            y = y * wg[j:j + 1, :]
            o = out_assign[j]
            accs[o] = (accs[o] + hs[sid] * w[j:j + 1, 1:2]
                       + y[:, 0:c] + y[:, c:2 * c]
                       + y[:, 2 * c:3 * c] + y[:, 3 * c:4 * c])
        o_refs[0][...] = accs[0]
        if two_out:
            o_refs[1][...] = accs[1]

    n_out = 2 if two_out else 1
    out = pl.pallas_call(
        body,
        grid=(g,),
        in_specs=in_specs,
        out_specs=[pl.BlockSpec((r, c), im_p0)] * n_out,
        out_shape=[jax.ShapeDtypeStruct((n, c), jnp.float32)] * n_out,
    )(*args)
    return out if two_out else (out[0], None)


def _classifier(parts, w0, wrest, b):
    n = parts[0].shape[0]
    ncls = wrest.shape[1]
    np_ = len(parts)

    def body(*refs):
        prefs = refs[:np_]
        w0_ref, wr_ref, b_ref, o_ref = refs[np_:]
        tot = 0.0
        acc = b_ref[...]
        off = 0
        for p in prefs:
            s = p[...]
            w = s.shape[1]
            tot = tot + jnp.sum(s, axis=1, keepdims=True)
            acc = acc + jnp.dot(s, wr_ref[pl.ds(off, w), :],
                                preferred_element_type=jnp.float32)
            off += w
        pooled = tot * (1.0 / off)
        o_ref[...] = acc + pooled * w0_ref[...]

    return pl.pallas_call(
        body, out_shape=jax.ShapeDtypeStruct((n, ncls), jnp.float32),
    )(*parts, w0, wrest, b)


def kernel(x, edge_index, stem_W, pre0_W0, pre1_W0, pre0_W1, pre1_W1,
           pre0_W2, pre1_W2, Wg, Wi, Ws, Wl, alphas, cls_W, cls_b):
    n = x.shape[0]
    c = Wg.shape[-1]
    src2 = 2 * edge_index[0]
    srcx = jnp.concatenate([src2, src2 + 1]).reshape(-1, _K)
    src2d = edge_index[0].reshape(-1, _K)
    dst2d = edge_index[1].reshape(-1, _K)

    zc = jnp.zeros_like(Wg)
    ws_h, ws_m = Ws[:, :, :c, :], Ws[:, :, c:, :]
    wbig = jnp.concatenate([
        jnp.concatenate([Wg, Wi, ws_h, Wl], axis=-1),
        jnp.concatenate([Wg, zc, ws_m, zc], axis=-1),
        jnp.concatenate([zc, Wi, zc, zc], axis=-1),
    ], axis=-2)

    stem = _stem(x, stem_W)
    s0_parts = [stem]
    s1_parts = [stem]
    pres = [(pre0_W0, pre1_W0), (pre0_W1, pre1_W1), (pre0_W2, pre1_W2)]
    degp = _deg_sc(dst2d, n)
    for cell in range(3):
        p0, p1 = pres[cell]
        h01 = _pre(s0_parts, s1_parts, p0, p1)
        mp01 = _segsum_sc(h01, srcx, dst2d, n, h01.shape[1],
                          split_edges=False)
        wb = wbig[cell]
        s2, part = _step2(h01, mp01, [], degp, wb[0:4], alphas[0:4],
                          out_assign=[0, 0, 1, 1], state_assign=[0, 1, 0, 1])
        mp2 = _segsum_sc(s2, src2d, dst2d, n, c, split_edges=True)
        s3, part = _step2(h01, mp01, [(s2, mp2)], degp, wb[4:8], alphas[4:8],
                          out_assign=[0, 1, 1, 1], state_assign=[2, 0, 1, 2],
                          base=part)
        mp3 = _segsum_sc(s3, src2d, dst2d, n, c, split_edges=True)
        s4, part = _step2(h01, mp01, [(s2, mp2), (s3, mp3)], degp,
                          wb[8:13], alphas[8:13],
                          out_assign=[0, 1, 1, 1, 1],
                          state_assign=[3, 0, 1, 2, 3], base=part)
        mp4 = _segsum_sc(s4, src2d, dst2d, n, c, split_edges=True)
        s5, _ = _step2(None, None, [(s4, mp4)], degp, wb[13:14],
                       alphas[13:14], out_assign=[0], state_assign=[0],
                       base=part)
        s0_parts, s1_parts = s1_parts, [s2, s3, s4, s5]

    return _classifier(s1_parts, cls_W[0:1], cls_W[1:], cls_b.reshape(1, -1))


# speedup vs baseline: 1.0251x; 1.0251x over previous
import functools

import jax
import jax.numpy as jnp
from jax import lax
from jax.experimental import pallas as pl
from jax.experimental.pallas import tpu as pltpu
from jax.experimental.pallas import tpu_sc as plsc

_NC, _NS = 2, 16
_K = 100
_BN_EPS = 1e-5


def _segsum_sc(h, src_idx, dst2d, n, d, split_edges):
    if split_edges:
        gw = d
        h2 = h
        nw = _NC * _NS
    else:
        gw = d // 2
        h2 = h.reshape(2 * n, gw)
        nw = _NS
    nch = dst2d.shape[0]
    cpw = nch // nw
    zr = 200
    nzc = n // zr
    ztrip = (nzc + _NS - 1) // _NS
    mesh = plsc.VectorSubcoreMesh(
        core_axis_name="c", subcore_axis_name="s",
        num_cores=_NC, num_subcores=_NS)

    @functools.partial(
        pl.kernel,
        out_type=jax.ShapeDtypeStruct((_NC * n, gw), jnp.float32),
        mesh=mesh,
        scratch_types=[
            pltpu.VMEM((cpw, _K), jnp.int32),
            pltpu.VMEM((cpw, _K), jnp.int32),
            [pltpu.VMEM((_K, gw), jnp.float32) for _ in range(4)],
            pltpu.VMEM((zr, gw), jnp.float32),
            pltpu.VMEM_SHARED((n, gw), jnp.float32),
            [pltpu.SemaphoreType.DMA for _ in range(4)],
        ],
        compiler_params=pltpu.CompilerParams(use_tc_tiling_on_sc=False),
    )
    def k(h_hbm, src_hbm, dst_hbm, out_hbm, srcb, dstb, bufs, zbuf, acc, sems):
        c = lax.axis_index("c")
        s = lax.axis_index("s")

        zeros16 = jnp.zeros((16,), jnp.float32)

        def zrow(i, _):
            def zcol(j, _):
                zbuf[i, pl.ds(j * 16, 16)] = zeros16
                return 0
            return lax.fori_loop(0, gw // 16, zcol, 0)

        lax.fori_loop(0, zr, zrow, 0)

        def zcopy(t, _):
            ch = jnp.minimum(s + t * _NS, nzc - 1)
            pltpu.sync_copy(zbuf, acc.at[pl.ds(ch * zr, zr)])
            return 0

        lax.fori_loop(0, ztrip, zcopy, 0)

        if split_edges:
            wid = s * _NC + c
            pltpu.sync_copy(src_hbm.at[pl.ds(wid * cpw, cpw)], srcb)
            pltpu.sync_copy(dst_hbm.at[pl.ds(wid * cpw, cpw)], dstb)
        else:
            pltpu.sync_copy(src_hbm.at[pl.ds(c * nch + s * cpw, cpw)], srcb)
            pltpu.sync_copy(dst_hbm.at[pl.ds(s * cpw, cpw)], dstb)
        plsc.subcore_barrier()

        for l in range(4):
            pltpu.async_copy(h_hbm.at[srcb.at[l]], bufs[l], sems[l])

        def body(i, _):
            for l in range(4):
                j = 4 * i + l
                pltpu.make_async_copy(
                    h_hbm.at[srcb.at[j]], bufs[l], sems[l]).wait()
                pltpu.sync_copy(bufs[l], acc.at[dstb.at[j]], add=True)
                jn = jnp.minimum(4 * i + 4 + l, cpw - 4 + l)
                pltpu.async_copy(h_hbm.at[srcb.at[jn]], bufs[l], sems[l])
            return 0

        lax.fori_loop(0, cpw // 4, body, 0)
        for l in range(4):
            pltpu.make_async_copy(
                h_hbm.at[srcb.at[l]], bufs[l], sems[l]).wait()
        plsc.subcore_barrier()

        def wcopy(t, _):
            ch = jnp.minimum(s + t * _NS, nzc - 1)
            pltpu.sync_copy(acc.at[pl.ds(ch * zr, zr)],
                            out_hbm.at[pl.ds(c * n + ch * zr, zr)])
            return 0

        lax.fori_loop(0, ztrip, wcopy, 0)

    return k(h2, src_idx, dst2d)


def _deg_sc(dst2d, n):
    nch = dst2d.shape[0]
    cpw = nch // (_NC * _NS)
    zr = 200
    nzc = n // zr
    ztrip = (nzc + _NS - 1) // _NS
    mesh = plsc.VectorSubcoreMesh(
        core_axis_name="c", subcore_axis_name="s",
        num_cores=_NC, num_subcores=_NS)

    @functools.partial(
        pl.kernel,
        out_type=jax.ShapeDtypeStruct((_NC * n, 16), jnp.float32),
        mesh=mesh,
        scratch_types=[
            pltpu.VMEM((cpw, _K), jnp.int32),
            pltpu.VMEM((_K, 16), jnp.float32),
            pltpu.VMEM((zr, 16), jnp.float32),
            pltpu.VMEM_SHARED((n, 16), jnp.float32),
        ],
        compiler_params=pltpu.CompilerParams(use_tc_tiling_on_sc=False),
    )
    def k(dst_hbm, out_hbm, dstb, ones_b, zbuf, acc):
        c = lax.axis_index("c")
        s = lax.axis_index("s")
        wid = s * _NC + c

        ones16 = jnp.ones((16,), jnp.float32)
        zeros16 = jnp.zeros((16,), jnp.float32)

        def orow(i, _):
            ones_b[i, pl.ds(0, 16)] = ones16
            return 0

        lax.fori_loop(0, _K, orow, 0)

        def zrow(i, _):
            zbuf[i, pl.ds(0, 16)] = zeros16
            return 0

        lax.fori_loop(0, zr, zrow, 0)

        def zcopy(t, _):
            ch = jnp.minimum(s + t * _NS, nzc - 1)
            pltpu.sync_copy(zbuf, acc.at[pl.ds(ch * zr, zr)])
            return 0

        lax.fori_loop(0, ztrip, zcopy, 0)

        pltpu.sync_copy(dst_hbm.at[pl.ds(wid * cpw, cpw)], dstb)
        plsc.subcore_barrier()

        def body(j, _):
            pltpu.sync_copy(ones_b, acc.at[dstb.at[j]], add=True)
            return 0

        lax.fori_loop(0, cpw, body, 0)
        plsc.subcore_barrier()

        def wcopy(t, _):
            ch = jnp.minimum(s + t * _NS, nzc - 1)
            pltpu.sync_copy(acc.at[pl.ds(ch * zr, zr)],
                            out_hbm.at[pl.ds(c * n + ch * zr, zr)])
            return 0

        lax.fori_loop(0, ztrip, wcopy, 0)

    return k(dst2d)


def _bn(u):
    mu = jnp.mean(u, axis=0, keepdims=True)
    var = jnp.mean((u - mu) * (u - mu), axis=0, keepdims=True)
    return (u - mu) / jnp.sqrt(var + _BN_EPS)


def _stem(x, stem_w):
    n = x.shape[0]
    m = stem_w.shape[1]

    def body(x_ref, w_ref, o_ref):
        u = jnp.dot(x_ref[...], w_ref[...], preferred_element_type=jnp.float32)
        o_ref[...] = _bn(u)

    return pl.pallas_call(
        body, out_shape=jax.ShapeDtypeStruct((n, m), jnp.float32),
    )(x, stem_w)


def _pre(s0_list, s1_list, p0, p1):
    n = s0_list[0].shape[0]
    c = p0.shape[1]
    n0, n1 = len(s0_list), len(s1_list)

    def body(*refs):
        s0r = refs[:n0]
        s1r = refs[n0:n0 + n1]
        p0r = refs[n0 + n1]
        p1r = refs[n0 + n1 + 1]
        o_ref = refs[n0 + n1 + 2]
        off = 0
        u0 = jnp.zeros((n, c), jnp.float32)
        for part in s0r:
            w = part.shape[1]
            u0 = u0 + jnp.dot(part[...], p0r[pl.ds(off, w), :],
                              preferred_element_type=jnp.float32)
            off += w
        off = 0
        u1 = jnp.zeros((n, c), jnp.float32)
        for part in s1r:
            w = part.shape[1]
            u1 = u1 + jnp.dot(part[...], p1r[pl.ds(off, w), :],
                              preferred_element_type=jnp.float32)
            off += w
        h0 = _bn(jax.nn.relu(u0))
        h1 = _bn(jax.nn.relu(u1))
        o_ref[...] = jnp.concatenate([h0, h1], axis=1)

    return pl.pallas_call(
        body, out_shape=jax.ShapeDtypeStruct((n, 2 * c), jnp.float32),
    )(*s0_list, *s1_list, p0, p1)


def _step2(h01, mp01, extras, degp, wb, al, out_assign, state_assign,
           base=None):
    c = wb.shape[1] // 3
    k = wb.shape[0]
    if h01 is not None:
        n, d01 = h01.shape
    else:
        n = extras[0][0].shape[0]
        d01 = 0
    r = 2000 if n % 2000 == 0 else n
    g = n // r
    two_out = any(o == 1 for o in out_assign)

    def im_p0(i):
        return (i, 0)

    def im_p1(i):
        return (i + g, 0)

    in_specs = []
    args = []
    if h01 is not None:
        in_specs += [
            pl.BlockSpec((r, d01), im_p0),
            pl.BlockSpec((r, c), im_p0),
            pl.BlockSpec((r, c), im_p1),
        ]
        args += [h01, mp01, mp01]
    for (s_j, mp_j) in extras:
        in_specs += [
            pl.BlockSpec((r, c), im_p0),
            pl.BlockSpec((r, c), im_p0),
            pl.BlockSpec((r, c), im_p1),
        ]
        args += [s_j, mp_j, mp_j]
    if base is not None:
        in_specs += [pl.BlockSpec((r, c), im_p0)]
        args += [base]
    in_specs += [
        pl.BlockSpec((r, 16), im_p0),
        pl.BlockSpec((r, 16), im_p1),
        pl.BlockSpec((k, 3 * c, 4 * c), lambda i: (0, 0, 0)),
        pl.BlockSpec((k, 8), lambda i: (0, 0)),
    ]
    args += [degp, degp, wb, jnp.pad(al, ((0, 0), (0, 2)))]
    nh = 3 if h01 is not None else 0
    nex = len(extras)
    nb = 1 if base is not None else 0
    nfirst = 2 if h01 is not None else 0

    def body(*refs):
        ex = refs[nh:nh + 3 * nex]
        base_ref = refs[nh + 3 * nex] if nb else None
        dg0, dg1, wb_ref, al_ref = refs[nh + 3 * nex + nb:
                                        nh + 3 * nex + nb + 4]
        o_refs = refs[nh + 3 * nex + nb + 4:]

        deg = dg0[:, 0:1] + dg1[:, 0:1]
        rdeg = 1.0 / jnp.maximum(deg, 1.0)
        alv = al_ref[...][:, 0:6]
        w = jax.nn.softmax(alv, axis=-1)
        wg = jnp.reshape(
            jnp.broadcast_to(w[:, 2:6][:, :, None], (k, 4, c)), (k, 4 * c))

        xcats = {}
        hs = {}
        for sid in sorted(set(state_assign)):
            if sid < nfirst:
                h = refs[0][:, sid * c:(sid + 1) * c]
                msum = refs[1][...] if sid == 0 else refs[2][...]
            else:
                e = sid - nfirst
                h = ex[3 * e][...]
                msum = ex[3 * e + 1][...] + ex[3 * e + 2][...]
            hs[sid] = h
            xcats[sid] = jnp.concatenate([h, msum * rdeg, msum], axis=1)

        accs = [base_ref[...] if nb else jnp.zeros((r, c), jnp.float32),
                jnp.zeros((r, c), jnp.float32)]
        for j in range(k):
            sid = state_assign[j]
            y = jax.nn.relu(jnp.dot(xcats[sid], wb_ref[j],
                                    preferred_element_type=jnp.float32))
            y = y * wg[j:j + 1, :]
            o = out_assign[j]
            accs[o] = (accs[o] + hs[sid] * w[j:j + 1, 1:2]
                       + y[:, 0:c] + y[:, c:2 * c]
                       + y[:, 2 * c:3 * c] + y[:, 3 * c:4 * c])
        o_refs[0][...] = accs[0]
        if two_out:
            o_refs[1][...] = accs[1]

    n_out = 2 if two_out else 1
    out = pl.pallas_call(
        body,
        grid=(g,),
        in_specs=in_specs,
        out_specs=[pl.BlockSpec((r, c), im_p0)] * n_out,
        out_shape=[jax.ShapeDtypeStruct((n, c), jnp.float32)] * n_out,
    )(*args)
    return out if two_out else (out[0], None)


def _classifier(parts, w0, wrest, b):
    n = parts[0].shape[0]
    ncls = wrest.shape[1]
    np_ = len(parts)

    def body(*refs):
        prefs = refs[:np_]
        w0_ref, wr_ref, b_ref, o_ref = refs[np_:]
        tot = 0.0
        acc = b_ref[...]
        off = 0
        for p in prefs:
            s = p[...]
            w = s.shape[1]
            tot = tot + jnp.sum(s, axis=1, keepdims=True)
            acc = acc + jnp.dot(s, wr_ref[pl.ds(off, w), :],
                                preferred_element_type=jnp.float32)
            off += w
        pooled = tot * (1.0 / off)
        o_ref[...] = acc + pooled * w0_ref[...]

    return pl.pallas_call(
        body, out_shape=jax.ShapeDtypeStruct((n, ncls), jnp.float32),
    )(*parts, w0, wrest, b)


def kernel(x, edge_index, stem_W, pre0_W0, pre1_W0, pre0_W1, pre1_W1,
           pre0_W2, pre1_W2, Wg, Wi, Ws, Wl, alphas, cls_W, cls_b):
    n = x.shape[0]
    c = Wg.shape[-1]
    src2 = 2 * edge_index[0]
    srcx = jnp.concatenate([src2, src2 + 1]).reshape(-1, _K)
    src2d = edge_index[0].reshape(-1, _K)
    dst2d = edge_index[1].reshape(-1, _K)

    zc = jnp.zeros_like(Wg)
    ws_h, ws_m = Ws[:, :, :c, :], Ws[:, :, c:, :]
    wbig = jnp.concatenate([
        jnp.concatenate([Wg, Wi, ws_h, Wl], axis=-1),
        jnp.concatenate([Wg, zc, ws_m, zc], axis=-1),
        jnp.concatenate([zc, Wi, zc, zc], axis=-1),
    ], axis=-2)

    stem = _stem(x, stem_W)
    s0_parts = [stem]
    s1_parts = [stem]
    pres = [(pre0_W0, pre1_W0), (pre0_W1, pre1_W1), (pre0_W2, pre1_W2)]
    degp = _deg_sc(dst2d, n)
    for cell in range(3):
        p0, p1 = pres[cell]
        h01 = _pre(s0_parts, s1_parts, p0, p1)
        mp01 = _segsum_sc(h01, srcx, dst2d, n, h01.shape[1],
                          split_edges=False)
        wb = wbig[cell]
        s2, _ = _step2(h01, mp01, [], degp, wb[0:2], alphas[0:2],
                       out_assign=[0, 0], state_assign=[0, 1])
        mp2 = _segsum_sc(s2, src2d, dst2d, n, c, split_edges=True)
        part, _ = _step2(h01, mp01, [], degp, wb[2:4], alphas[2:4],
                         out_assign=[0, 0], state_assign=[0, 1])
        s3, _ = _step2(None, None, [(s2, mp2)], degp, wb[4:5], alphas[4:5],
                       out_assign=[0], state_assign=[0], base=part)
        mp3 = _segsum_sc(s3, src2d, dst2d, n, c, split_edges=True)
        part, _ = _step2(h01, mp01, [(s2, mp2)], degp, wb[5:8], alphas[5:8],
                         out_assign=[0, 0, 0], state_assign=[0, 1, 2])
        s4, _ = _step2(None, None, [(s3, mp3)], degp, wb[8:9], alphas[8:9],
                       out_assign=[0], state_assign=[0], base=part)
        mp4 = _segsum_sc(s4, src2d, dst2d, n, c, split_edges=True)
        part, _ = _step2(h01, mp01, [(s2, mp2), (s3, mp3)], degp,
                         wb[9:13], alphas[9:13],
                         out_assign=[0, 0, 0, 0], state_assign=[0, 1, 2, 3])
        s5, _ = _step2(None, None, [(s4, mp4)], degp, wb[13:14],
                       alphas[13:14], out_assign=[0], state_assign=[0],
                       base=part)
        s0_parts, s1_parts = s1_parts, [s2, s3, s4, s5]

    return _classifier(s1_parts, cls_W[0:1], cls_W[1:], cls_b.reshape(1, -1))


# speedup vs baseline: 1.0372x; 1.0119x over previous
import functools

import jax
import jax.numpy as jnp
from jax import lax
from jax.experimental import pallas as pl
from jax.experimental.pallas import tpu as pltpu
from jax.experimental.pallas import tpu_sc as plsc

_NC, _NS = 2, 16
_K = 100
_BN_EPS = 1e-5


def _segsum_sc(h, src_idx, dst2d, n, d, split_edges):
    if split_edges:
        gw = d
        h2 = h
        nw = _NC * _NS
    else:
        gw = d // 2
        h2 = h.reshape(2 * n, gw)
        nw = _NS
    nch = dst2d.shape[0]
    cpw = nch // nw
    zr = 200
    nzc = n // zr
    ztrip = (nzc + _NS - 1) // _NS
    mesh = plsc.VectorSubcoreMesh(
        core_axis_name="c", subcore_axis_name="s",
        num_cores=_NC, num_subcores=_NS)

    @functools.partial(
        pl.kernel,
        out_type=jax.ShapeDtypeStruct((_NC * n, gw), jnp.float32),
        mesh=mesh,
        scratch_types=[
            pltpu.VMEM((cpw, _K), jnp.int32),
            pltpu.VMEM((cpw, _K), jnp.int32),
            [pltpu.VMEM((_K, gw), jnp.float32) for _ in range(4)],
            pltpu.VMEM((zr, gw), jnp.float32),
            pltpu.VMEM_SHARED((n, gw), jnp.float32),
            [pltpu.SemaphoreType.DMA for _ in range(4)],
        ],
        compiler_params=pltpu.CompilerParams(use_tc_tiling_on_sc=False),
    )
    def k(h_hbm, src_hbm, dst_hbm, out_hbm, srcb, dstb, bufs, zbuf, acc, sems):
        c = lax.axis_index("c")
        s = lax.axis_index("s")

        zeros16 = jnp.zeros((16,), jnp.float32)

        def zrow(i, _):
            def zcol(j, _):
                zbuf[i, pl.ds(j * 16, 16)] = zeros16
                return 0
            return lax.fori_loop(0, gw // 16, zcol, 0)

        lax.fori_loop(0, zr, zrow, 0)

        def zcopy(t, _):
            ch = jnp.minimum(s + t * _NS, nzc - 1)
            pltpu.sync_copy(zbuf, acc.at[pl.ds(ch * zr, zr)])
            return 0

        lax.fori_loop(0, ztrip, zcopy, 0)

        if split_edges:
            wid = s * _NC + c
            pltpu.sync_copy(src_hbm.at[pl.ds(wid * cpw, cpw)], srcb)
            pltpu.sync_copy(dst_hbm.at[pl.ds(wid * cpw, cpw)], dstb)
        else:
            pltpu.sync_copy(src_hbm.at[pl.ds(c * nch + s * cpw, cpw)], srcb)
            pltpu.sync_copy(dst_hbm.at[pl.ds(s * cpw, cpw)], dstb)
        plsc.subcore_barrier()

        for l in range(4):
            pltpu.async_copy(h_hbm.at[srcb.at[l]], bufs[l], sems[l])

        def body(i, _):
            for l in range(4):
                j = 4 * i + l
                pltpu.make_async_copy(
                    h_hbm.at[srcb.at[j]], bufs[l], sems[l]).wait()
                pltpu.sync_copy(bufs[l], acc.at[dstb.at[j]], add=True)
                jn = jnp.minimum(4 * i + 4 + l, cpw - 4 + l)
                pltpu.async_copy(h_hbm.at[srcb.at[jn]], bufs[l], sems[l])
            return 0

        lax.fori_loop(0, cpw // 4, body, 0)
        for l in range(4):
            pltpu.make_async_copy(
                h_hbm.at[srcb.at[l]], bufs[l], sems[l]).wait()
        plsc.subcore_barrier()

        def wcopy(t, _):
            ch = jnp.minimum(s + t * _NS, nzc - 1)
            pltpu.sync_copy(acc.at[pl.ds(ch * zr, zr)],
                            out_hbm.at[pl.ds(c * n + ch * zr, zr)])
            return 0

        lax.fori_loop(0, ztrip, wcopy, 0)

    return k(h2, src_idx, dst2d)


def _deg_sc(dst2d, n):
    nch = dst2d.shape[0]
    cpw = nch // (_NC * _NS)
    zr = 200
    nzc = n // zr
    ztrip = (nzc + _NS - 1) // _NS
    mesh = plsc.VectorSubcoreMesh(
        core_axis_name="c", subcore_axis_name="s",
        num_cores=_NC, num_subcores=_NS)

    @functools.partial(
        pl.kernel,
        out_type=jax.ShapeDtypeStruct((_NC * n, 16), jnp.float32),
        mesh=mesh,
        scratch_types=[
            pltpu.VMEM((cpw, _K), jnp.int32),
            pltpu.VMEM((_K, 16), jnp.float32),
            pltpu.VMEM((zr, 16), jnp.float32),
            pltpu.VMEM_SHARED((n, 16), jnp.float32),
        ],
        compiler_params=pltpu.CompilerParams(use_tc_tiling_on_sc=False),
    )
    def k(dst_hbm, out_hbm, dstb, ones_b, zbuf, acc):
        c = lax.axis_index("c")
        s = lax.axis_index("s")
        wid = s * _NC + c

        ones16 = jnp.ones((16,), jnp.float32)
        zeros16 = jnp.zeros((16,), jnp.float32)

        def orow(i, _):
            ones_b[i, pl.ds(0, 16)] = ones16
            return 0

        lax.fori_loop(0, _K, orow, 0)

        def zrow(i, _):
            zbuf[i, pl.ds(0, 16)] = zeros16
            return 0

        lax.fori_loop(0, zr, zrow, 0)

        def zcopy(t, _):
            ch = jnp.minimum(s + t * _NS, nzc - 1)
            pltpu.sync_copy(zbuf, acc.at[pl.ds(ch * zr, zr)])
            return 0

        lax.fori_loop(0, ztrip, zcopy, 0)

        pltpu.sync_copy(dst_hbm.at[pl.ds(wid * cpw, cpw)], dstb)
        plsc.subcore_barrier()

        def body(j, _):
            pltpu.sync_copy(ones_b, acc.at[dstb.at[j]], add=True)
            return 0

        lax.fori_loop(0, cpw, body, 0)
        plsc.subcore_barrier()

        def wcopy(t, _):
            ch = jnp.minimum(s + t * _NS, nzc - 1)
            pltpu.sync_copy(acc.at[pl.ds(ch * zr, zr)],
                            out_hbm.at[pl.ds(c * n + ch * zr, zr)])
            return 0

        lax.fori_loop(0, ztrip, wcopy, 0)

    return k(dst2d)


def _bn(u):
    mu = jnp.mean(u, axis=0, keepdims=True)
    var = jnp.mean((u - mu) * (u - mu), axis=0, keepdims=True)
    return (u - mu) / jnp.sqrt(var + _BN_EPS)


def _stem(x, stem_w):
    n = x.shape[0]
    m = stem_w.shape[1]

    def body(x_ref, w_ref, o_ref):
        u = jnp.dot(x_ref[...], w_ref[...], preferred_element_type=jnp.float32)
        o_ref[...] = _bn(u)

    return pl.pallas_call(
        body, out_shape=jax.ShapeDtypeStruct((n, m), jnp.float32),
    )(x, stem_w)


def _pre(s0_list, s1_list, p0, p1):
    n = s0_list[0].shape[0]
    c = p0.shape[1]
    n0, n1 = len(s0_list), len(s1_list)

    def body(*refs):
        s0r = refs[:n0]
        s1r = refs[n0:n0 + n1]
        p0r = refs[n0 + n1]
        p1r = refs[n0 + n1 + 1]
        o_ref = refs[n0 + n1 + 2]
        off = 0
        u0 = jnp.zeros((n, c), jnp.float32)
        for part in s0r:
            w = part.shape[1]
            u0 = u0 + jnp.dot(part[...], p0r[pl.ds(off, w), :],
                              preferred_element_type=jnp.float32)
            off += w
        off = 0
        u1 = jnp.zeros((n, c), jnp.float32)
        for part in s1r:
            w = part.shape[1]
            u1 = u1 + jnp.dot(part[...], p1r[pl.ds(off, w), :],
                              preferred_element_type=jnp.float32)
            off += w
        h0 = _bn(jax.nn.relu(u0))
        h1 = _bn(jax.nn.relu(u1))
        o_ref[...] = jnp.concatenate([h0, h1], axis=1)

    return pl.pallas_call(
        body, out_shape=jax.ShapeDtypeStruct((n, 2 * c), jnp.float32),
    )(*s0_list, *s1_list, p0, p1)


def _step2(h01, mp01, extras, degp, wb, al, out_assign, state_assign,
           base=None):
    c = wb.shape[1] // 3
    k = wb.shape[0]
    if h01 is not None:
        n, d01 = h01.shape
    else:
        n = extras[0][0].shape[0]
        d01 = 0
    r = 2000 if n % 2000 == 0 else n
    g = n // r
    two_out = any(o == 1 for o in out_assign)

    def im_p0(i):
        return (i, 0)

    def im_p1(i):
        return (i + g, 0)

    in_specs = []
    args = []
    if h01 is not None:
        in_specs += [
            pl.BlockSpec((r, d01), im_p0),
            pl.BlockSpec((r, c), im_p0),
            pl.BlockSpec((r, c), im_p1),
        ]
        args += [h01, mp01, mp01]
    for (s_j, mp_j) in extras:
        in_specs += [
            pl.BlockSpec((r, c), im_p0),
            pl.BlockSpec((r, c), im_p0),
            pl.BlockSpec((r, c), im_p1),
        ]
        args += [s_j, mp_j, mp_j]
    if base is not None:
        in_specs += [pl.BlockSpec((r, c), im_p0)]
        args += [base]
    in_specs += [
        pl.BlockSpec((r, 16), im_p0),
        pl.BlockSpec((r, 16), im_p1),
        pl.BlockSpec((k, 3 * c, 4 * c), lambda i: (0, 0, 0)),
        pl.BlockSpec((k, 8), lambda i: (0, 0)),
    ]
    args += [degp, degp, wb, jnp.pad(al, ((0, 0), (0, 2)))]
    nh = 3 if h01 is not None else 0
    nex = len(extras)
    nb = 1 if base is not None else 0
    nfirst = 2 if h01 is not None else 0

    def body(*refs):
        ex = refs[nh:nh + 3 * nex]
        base_ref = refs[nh + 3 * nex] if nb else None
        dg0, dg1, wb_ref, al_ref = refs[nh + 3 * nex + nb:
                                        nh + 3 * nex + nb + 4]
        o_refs = refs[nh + 3 * nex + nb + 4:]

        deg = dg0[:, 0:1] + dg1[:, 0:1]
        rdeg = 1.0 / jnp.maximum(deg, 1.0)
        alv = al_ref[...][:, 0:6]
        w = jax.nn.softmax(alv, axis=-1)
        wg = jnp.reshape(
            jnp.broadcast_to(w[:, 2:6][:, :, None], (k, 4, c)), (k, 4 * c))

        xcats = {}
        hs = {}
        for sid in sorted(set(state_assign)):
            if sid < nfirst:
                h = refs[0][:, sid * c:(sid + 1) * c]
                msum = refs[1][...] if sid == 0 else refs[2][...]
            else:
                e = sid - nfirst
                h = ex[3 * e][...]
                msum = ex[3 * e + 1][...] + ex[3 * e + 2][...]
            hs[sid] = h
            xcats[sid] = jnp.concatenate(
                [h, msum * rdeg, msum], axis=1).astype(jnp.bfloat16)

        accs = [base_ref[...] if nb else jnp.zeros((r, c), jnp.float32),
                jnp.zeros((r, c), jnp.float32)]
        for j in range(k):
            sid = state_assign[j]
            y = jax.nn.relu(jnp.dot(xcats[sid], wb_ref[j],
                                    preferred_element_type=jnp.float32))
            y = y * wg[j:j + 1, :]
            o = out_assign[j]
            accs[o] = (accs[o] + hs[sid] * w[j:j + 1, 1:2]
                       + y[:, 0:c] + y[:, c:2 * c]
                       + y[:, 2 * c:3 * c] + y[:, 3 * c:4 * c])
        o_refs[0][...] = accs[0]
        if two_out:
            o_refs[1][...] = accs[1]

    n_out = 2 if two_out else 1
    out = pl.pallas_call(
        body,
        grid=(g,),
        in_specs=in_specs,
        out_specs=[pl.BlockSpec((r, c), im_p0)] * n_out,
        out_shape=[jax.ShapeDtypeStruct((n, c), jnp.float32)] * n_out,
    )(*args)
    return out if two_out else (out[0], None)


def _classifier(parts, w0, wrest, b):
    n = parts[0].shape[0]
    ncls = wrest.shape[1]
    np_ = len(parts)

    def body(*refs):
        prefs = refs[:np_]
        w0_ref, wr_ref, b_ref, o_ref = refs[np_:]
        tot = 0.0
        acc = b_ref[...]
        off = 0
        for p in prefs:
            s = p[...]
            w = s.shape[1]
            tot = tot + jnp.sum(s, axis=1, keepdims=True)
            acc = acc + jnp.dot(s, wr_ref[pl.ds(off, w), :],
                                preferred_element_type=jnp.float32)
            off += w
        pooled = tot * (1.0 / off)
        o_ref[...] = acc + pooled * w0_ref[...]

    return pl.pallas_call(
        body, out_shape=jax.ShapeDtypeStruct((n, ncls), jnp.float32),
    )(*parts, w0, wrest, b)


def kernel(x, edge_index, stem_W, pre0_W0, pre1_W0, pre0_W1, pre1_W1,
           pre0_W2, pre1_W2, Wg, Wi, Ws, Wl, alphas, cls_W, cls_b):
    n = x.shape[0]
    c = Wg.shape[-1]
    src2 = 2 * edge_index[0]
    srcx = jnp.concatenate([src2, src2 + 1]).reshape(-1, _K)
    src2d = edge_index[0].reshape(-1, _K)
    dst2d = edge_index[1].reshape(-1, _K)

    zc = jnp.zeros_like(Wg)
    ws_h, ws_m = Ws[:, :, :c, :], Ws[:, :, c:, :]
    wbig = jnp.concatenate([
        jnp.concatenate([Wg, Wi, ws_h, Wl], axis=-1),
        jnp.concatenate([Wg, zc, ws_m, zc], axis=-1),
        jnp.concatenate([zc, Wi, zc, zc], axis=-1),
    ], axis=-2).astype(jnp.bfloat16)

    stem = _stem(x, stem_W)
    s0_parts = [stem]
    s1_parts = [stem]
    pres = [(pre0_W0, pre1_W0), (pre0_W1, pre1_W1), (pre0_W2, pre1_W2)]
    degp = _deg_sc(dst2d, n)
    for cell in range(3):
        p0, p1 = pres[cell]
        h01 = _pre(s0_parts, s1_parts, p0, p1)
        mp01 = _segsum_sc(h01, srcx, dst2d, n, h01.shape[1],
                          split_edges=False)
        wb = wbig[cell]
        s2, _ = _step2(h01, mp01, [], degp, wb[0:2], alphas[0:2],
                       out_assign=[0, 0], state_assign=[0, 1])
        mp2 = _segsum_sc(s2, src2d, dst2d, n, c, split_edges=True)
        part, _ = _step2(h01, mp01, [], degp, wb[2:4], alphas[2:4],
                         out_assign=[0, 0], state_assign=[0, 1])
        s3, _ = _step2(None, None, [(s2, mp2)], degp, wb[4:5], alphas[4:5],
                       out_assign=[0], state_assign=[0], base=part)
        mp3 = _segsum_sc(s3, src2d, dst2d, n, c, split_edges=True)
        part, _ = _step2(h01, mp01, [(s2, mp2)], degp, wb[5:8], alphas[5:8],
                         out_assign=[0, 0, 0], state_assign=[0, 1, 2])
        s4, _ = _step2(None, None, [(s3, mp3)], degp, wb[8:9], alphas[8:9],
                       out_assign=[0], state_assign=[0], base=part)
        mp4 = _segsum_sc(s4, src2d, dst2d, n, c, split_edges=True)
        part, _ = _step2(h01, mp01, [(s2, mp2), (s3, mp3)], degp,
                         wb[9:13], alphas[9:13],
                         out_assign=[0, 0, 0, 0], state_assign=[0, 1, 2, 3])
        s5, _ = _step2(None, None, [(s4, mp4)], degp, wb[13:14],
                       alphas[13:14], out_assign=[0], state_assign=[0],
                       base=part)
        s0_parts, s1_parts = s1_parts, [s2, s3, s4, s5]

    return _classifier(s1_parts, cls_W[0:1], cls_W[1:], cls_b.reshape(1, -1))


# speedup vs baseline: 1.0453x; 1.0078x over previous
import functools

import jax
import jax.numpy as jnp
from jax import lax
from jax.experimental import pallas as pl
from jax.experimental.pallas import tpu as pltpu
from jax.experimental.pallas import tpu_sc as plsc

_NC, _NS = 2, 16
_K = 125
_BN_EPS = 1e-5


def _segsum_sc(h, src_idx, dst2d, n, d, split_edges):
    if split_edges:
        gw = d
        h2 = h
        nw = _NC * _NS
        nbuf = 5
    else:
        gw = d // 2
        h2 = h.reshape(2 * n, gw)
        nw = _NS
        nbuf = 4
    nch = dst2d.shape[0]
    cpw = nch // nw
    zr = 200
    nzc = n // zr
    ztrip = (nzc + _NS - 1) // _NS
    mesh = plsc.VectorSubcoreMesh(
        core_axis_name="c", subcore_axis_name="s",
        num_cores=_NC, num_subcores=_NS)

    @functools.partial(
        pl.kernel,
        out_type=jax.ShapeDtypeStruct((_NC * n, gw), jnp.float32),
        mesh=mesh,
        scratch_types=[
            pltpu.VMEM((cpw, _K), jnp.int32),
            pltpu.VMEM((cpw, _K), jnp.int32),
            [pltpu.VMEM((_K, gw), jnp.float32) for _ in range(nbuf)],
            pltpu.VMEM((zr, gw), jnp.float32),
            pltpu.VMEM_SHARED((n, gw), jnp.float32),
            [pltpu.SemaphoreType.DMA for _ in range(nbuf)],
        ],
        compiler_params=pltpu.CompilerParams(use_tc_tiling_on_sc=False),
    )
    def k(h_hbm, src_hbm, dst_hbm, out_hbm, srcb, dstb, bufs, zbuf, acc, sems):
        c = lax.axis_index("c")
        s = lax.axis_index("s")

        zeros16 = jnp.zeros((16,), jnp.float32)

        def zrow(i, _):
            def zcol(j, _):
                zbuf[i, pl.ds(j * 16, 16)] = zeros16
                return 0
            return lax.fori_loop(0, gw // 16, zcol, 0)

        lax.fori_loop(0, zr, zrow, 0)

        def zcopy(t, _):
            ch = jnp.minimum(s + t * _NS, nzc - 1)
            pltpu.sync_copy(zbuf, acc.at[pl.ds(ch * zr, zr)])
            return 0

        lax.fori_loop(0, ztrip, zcopy, 0)

        if split_edges:
            wid = s * _NC + c
            pltpu.sync_copy(src_hbm.at[pl.ds(wid * cpw, cpw)], srcb)
            pltpu.sync_copy(dst_hbm.at[pl.ds(wid * cpw, cpw)], dstb)
        else:
            pltpu.sync_copy(src_hbm.at[pl.ds(c * nch + s * cpw, cpw)], srcb)
            pltpu.sync_copy(dst_hbm.at[pl.ds(s * cpw, cpw)], dstb)
        plsc.subcore_barrier()

        for l in range(nbuf):
            pltpu.async_copy(h_hbm.at[srcb.at[l]], bufs[l], sems[l])

        def body(i, _):
            for l in range(nbuf):
                j = nbuf * i + l
                pltpu.make_async_copy(
                    h_hbm.at[srcb.at[j]], bufs[l], sems[l]).wait()
                pltpu.sync_copy(bufs[l], acc.at[dstb.at[j]], add=True)
                jn = jnp.minimum(nbuf * i + nbuf + l, cpw - nbuf + l)
                pltpu.async_copy(h_hbm.at[srcb.at[jn]], bufs[l], sems[l])
            return 0

        lax.fori_loop(0, cpw // nbuf, body, 0)
        for l in range(nbuf):
            pltpu.make_async_copy(
                h_hbm.at[srcb.at[l]], bufs[l], sems[l]).wait()
        plsc.subcore_barrier()

        def wcopy(t, _):
            ch = jnp.minimum(s + t * _NS, nzc - 1)
            pltpu.sync_copy(acc.at[pl.ds(ch * zr, zr)],
                            out_hbm.at[pl.ds(c * n + ch * zr, zr)])
            return 0

        lax.fori_loop(0, ztrip, wcopy, 0)

    return k(h2, src_idx, dst2d)


def _deg_sc(dst2d, n):
    nch = dst2d.shape[0]
    cpw = nch // (_NC * _NS)
    zr = 200
    nzc = n // zr
    ztrip = (nzc + _NS - 1) // _NS
    mesh = plsc.VectorSubcoreMesh(
        core_axis_name="c", subcore_axis_name="s",
        num_cores=_NC, num_subcores=_NS)

    @functools.partial(
        pl.kernel,
        out_type=jax.ShapeDtypeStruct((_NC * n, 16), jnp.float32),
        mesh=mesh,
        scratch_types=[
            pltpu.VMEM((cpw, _K), jnp.int32),
            pltpu.VMEM((_K, 16), jnp.float32),
            pltpu.VMEM((zr, 16), jnp.float32),
            pltpu.VMEM_SHARED((n, 16), jnp.float32),
        ],
        compiler_params=pltpu.CompilerParams(use_tc_tiling_on_sc=False),
    )
    def k(dst_hbm, out_hbm, dstb, ones_b, zbuf, acc):
        c = lax.axis_index("c")
        s = lax.axis_index("s")
        wid = s * _NC + c

        ones16 = jnp.ones((16,), jnp.float32)
        zeros16 = jnp.zeros((16,), jnp.float32)

        def orow(i, _):
            ones_b[i, pl.ds(0, 16)] = ones16
            return 0

        lax.fori_loop(0, _K, orow, 0)

        def zrow(i, _):
            zbuf[i, pl.ds(0, 16)] = zeros16
            return 0

        lax.fori_loop(0, zr, zrow, 0)

        def zcopy(t, _):
            ch = jnp.minimum(s + t * _NS, nzc - 1)
            pltpu.sync_copy(zbuf, acc.at[pl.ds(ch * zr, zr)])
            return 0

        lax.fori_loop(0, ztrip, zcopy, 0)

        pltpu.sync_copy(dst_hbm.at[pl.ds(wid * cpw, cpw)], dstb)
        plsc.subcore_barrier()

        def body(j, _):
            pltpu.sync_copy(ones_b, acc.at[dstb.at[j]], add=True)
            return 0

        lax.fori_loop(0, cpw, body, 0)
        plsc.subcore_barrier()

        def wcopy(t, _):
            ch = jnp.minimum(s + t * _NS, nzc - 1)
            pltpu.sync_copy(acc.at[pl.ds(ch * zr, zr)],
                            out_hbm.at[pl.ds(c * n + ch * zr, zr)])
            return 0

        lax.fori_loop(0, ztrip, wcopy, 0)

    return k(dst2d)


def _bn(u):
    mu = jnp.mean(u, axis=0, keepdims=True)
    var = jnp.mean((u - mu) * (u - mu), axis=0, keepdims=True)
    return (u - mu) / jnp.sqrt(var + _BN_EPS)


def _stem(x, stem_w):
    n = x.shape[0]
    m = stem_w.shape[1]

    def body(x_ref, w_ref, o_ref):
        u = jnp.dot(x_ref[...], w_ref[...], preferred_element_type=jnp.float32)
        o_ref[...] = _bn(u)

    return pl.pallas_call(
        body, out_shape=jax.ShapeDtypeStruct((n, m), jnp.float32),
    )(x, stem_w)


def _pre(s0_list, s1_list, p0, p1):
    n = s0_list[0].shape[0]
    c = p0.shape[1]
    n0, n1 = len(s0_list), len(s1_list)

    def body(*refs):
        s0r = refs[:n0]
        s1r = refs[n0:n0 + n1]
        p0r = refs[n0 + n1]
        p1r = refs[n0 + n1 + 1]
        o_ref = refs[n0 + n1 + 2]
        off = 0
        u0 = jnp.zeros((n, c), jnp.float32)
        for part in s0r:
            w = part.shape[1]
            u0 = u0 + jnp.dot(part[...], p0r[pl.ds(off, w), :],
                              preferred_element_type=jnp.float32)
            off += w
        off = 0
        u1 = jnp.zeros((n, c), jnp.float32)
        for part in s1r:
            w = part.shape[1]
            u1 = u1 + jnp.dot(part[...], p1r[pl.ds(off, w), :],
                              preferred_element_type=jnp.float32)
            off += w
        h0 = _bn(jax.nn.relu(u0))
        h1 = _bn(jax.nn.relu(u1))
        o_ref[...] = jnp.concatenate([h0, h1], axis=1)

    return pl.pallas_call(
        body, out_shape=jax.ShapeDtypeStruct((n, 2 * c), jnp.float32),
    )(*s0_list, *s1_list, p0, p1)


def _step2(h01, mp01, extras, degp, wb, al, out_assign, state_assign,
           base=None):
    c = wb.shape[1] // 3
    k = wb.shape[0]
    if h01 is not None:
        n, d01 = h01.shape
    else:
        n = extras[0][0].shape[0]
        d01 = 0
    r = 2000 if n % 2000 == 0 else n
    g = n // r
    two_out = any(o == 1 for o in out_assign)

    def im_p0(i):
        return (i, 0)

    def im_p1(i):
        return (i + g, 0)

    in_specs = []
    args = []
    if h01 is not None:
        in_specs += [
            pl.BlockSpec((r, d01), im_p0),
            pl.BlockSpec((r, c), im_p0),
            pl.BlockSpec((r, c), im_p1),
        ]
        args += [h01, mp01, mp01]
    for (s_j, mp_j) in extras:
        in_specs += [
            pl.BlockSpec((r, c), im_p0),
            pl.BlockSpec((r, c), im_p0),
            pl.BlockSpec((r, c), im_p1),
        ]
        args += [s_j, mp_j, mp_j]
    if base is not None:
        in_specs += [pl.BlockSpec((r, c), im_p0)]
        args += [base]
    in_specs += [
        pl.BlockSpec((r, 16), im_p0),
        pl.BlockSpec((r, 16), im_p1),
        pl.BlockSpec((k, 3 * c, 4 * c), lambda i: (0, 0, 0)),
        pl.BlockSpec((k, 8), lambda i: (0, 0)),
    ]
    args += [degp, degp, wb, jnp.pad(al, ((0, 0), (0, 2)))]
    nh = 3 if h01 is not None else 0
    nex = len(extras)
    nb = 1 if base is not None else 0
    nfirst = 2 if h01 is not None else 0

    def body(*refs):
        ex = refs[nh:nh + 3 * nex]
        base_ref = refs[nh + 3 * nex] if nb else None
        dg0, dg1, wb_ref, al_ref = refs[nh + 3 * nex + nb:
                                        nh + 3 * nex + nb + 4]
        o_refs = refs[nh + 3 * nex + nb + 4:]

        deg = dg0[:, 0:1] + dg1[:, 0:1]
        rdeg = 1.0 / jnp.maximum(deg, 1.0)
        alv = al_ref[...][:, 0:6]
        w = jax.nn.softmax(alv, axis=-1)
        wg = jnp.reshape(
            jnp.broadcast_to(w[:, 2:6][:, :, None], (k, 4, c)), (k, 4 * c))

        xcats = {}
        hs = {}
        for sid in sorted(set(state_assign)):
            if sid < nfirst:
                h = refs[0][:, sid * c:(sid + 1) * c]
                msum = refs[1][...] if sid == 0 else refs[2][...]
            else:
                e = sid - nfirst
                h = ex[3 * e][...]
                msum = ex[3 * e + 1][...] + ex[3 * e + 2][...]
            hs[sid] = h
            xcats[sid] = jnp.concatenate(
                [h, msum * rdeg, msum], axis=1).astype(jnp.bfloat16)

        accs = [base_ref[...] if nb else jnp.zeros((r, c), jnp.float32),
                jnp.zeros((r, c), jnp.float32)]
        for j in range(k):
            sid = state_assign[j]
            y = jax.nn.relu(jnp.dot(xcats[sid], wb_ref[j],
                                    preferred_element_type=jnp.float32))
            y = y * wg[j:j + 1, :]
            o = out_assign[j]
            accs[o] = (accs[o] + hs[sid] * w[j:j + 1, 1:2]
                       + y[:, 0:c] + y[:, c:2 * c]
                       + y[:, 2 * c:3 * c] + y[:, 3 * c:4 * c])
        o_refs[0][...] = accs[0]
        if two_out:
            o_refs[1][...] = accs[1]

    n_out = 2 if two_out else 1
    out = pl.pallas_call(
        body,
        grid=(g,),
        in_specs=in_specs,
        out_specs=[pl.BlockSpec((r, c), im_p0)] * n_out,
        out_shape=[jax.ShapeDtypeStruct((n, c), jnp.float32)] * n_out,
    )(*args)
    return out if two_out else (out[0], None)


def _classifier(parts, w0, wrest, b):
    n = parts[0].shape[0]
    ncls = wrest.shape[1]
    np_ = len(parts)

    def body(*refs):
        prefs = refs[:np_]
        w0_ref, wr_ref, b_ref, o_ref = refs[np_:]
        tot = 0.0
        acc = b_ref[...]
        off = 0
        for p in prefs:
            s = p[...]
            w = s.shape[1]
            tot = tot + jnp.sum(s, axis=1, keepdims=True)
            acc = acc + jnp.dot(s, wr_ref[pl.ds(off, w), :],
                                preferred_element_type=jnp.float32)
            off += w
        pooled = tot * (1.0 / off)
        o_ref[...] = acc + pooled * w0_ref[...]

    return pl.pallas_call(
        body, out_shape=jax.ShapeDtypeStruct((n, ncls), jnp.float32),
    )(*parts, w0, wrest, b)


def kernel(x, edge_index, stem_W, pre0_W0, pre1_W0, pre0_W1, pre1_W1,
           pre0_W2, pre1_W2, Wg, Wi, Ws, Wl, alphas, cls_W, cls_b):
    n = x.shape[0]
    c = Wg.shape[-1]
    src2 = 2 * edge_index[0]
    srcx = jnp.concatenate([src2, src2 + 1]).reshape(-1, _K)
    src2d = edge_index[0].reshape(-1, _K)
    dst2d = edge_index[1].reshape(-1, _K)

    zc = jnp.zeros_like(Wg)
    ws_h, ws_m = Ws[:, :, :c, :], Ws[:, :, c:, :]
    wbig = jnp.concatenate([
        jnp.concatenate([Wg, Wi, ws_h, Wl], axis=-1),
        jnp.concatenate([Wg, zc, ws_m, zc], axis=-1),
        jnp.concatenate([zc, Wi, zc, zc], axis=-1),
    ], axis=-2).astype(jnp.bfloat16)

    stem = _stem(x, stem_W)
    s0_parts = [stem]
    s1_parts = [stem]
    pres = [(pre0_W0, pre1_W0), (pre0_W1, pre1_W1), (pre0_W2, pre1_W2)]
    degp = _deg_sc(dst2d, n)
    for cell in range(3):
        p0, p1 = pres[cell]
        h01 = _pre(s0_parts, s1_parts, p0, p1)
        mp01 = _segsum_sc(h01, srcx, dst2d, n, h01.shape[1],
                          split_edges=False)
        wb = wbig[cell]
        s2, _ = _step2(h01, mp01, [], degp, wb[0:2], alphas[0:2],
                       out_assign=[0, 0], state_assign=[0, 1])
        mp2 = _segsum_sc(s2, src2d, dst2d, n, c, split_edges=True)
        part, _ = _step2(h01, mp01, [], degp, wb[2:4], alphas[2:4],
                         out_assign=[0, 0], state_assign=[0, 1])
        s3, _ = _step2(None, None, [(s2, mp2)], degp, wb[4:5], alphas[4:5],
                       out_assign=[0], state_assign=[0], base=part)
        mp3 = _segsum_sc(s3, src2d, dst2d, n, c, split_edges=True)
        part, _ = _step2(h01, mp01, [(s2, mp2)], degp, wb[5:8], alphas[5:8],
                         out_assign=[0, 0, 0], state_assign=[0, 1, 2])
        s4, _ = _step2(None, None, [(s3, mp3)], degp, wb[8:9], alphas[8:9],
                       out_assign=[0], state_assign=[0], base=part)
        mp4 = _segsum_sc(s4, src2d, dst2d, n, c, split_edges=True)
        part, _ = _step2(h01, mp01, [(s2, mp2), (s3, mp3)], degp,
                         wb[9:13], alphas[9:13],
                         out_assign=[0, 0, 0, 0], state_assign=[0, 1, 2, 3])
        s5, _ = _step2(None, None, [(s4, mp4)], degp, wb[13:14],
                       alphas[13:14], out_assign=[0], state_assign=[0],
                       base=part)
        s0_parts, s1_parts = s1_parts, [s2, s3, s4, s5]

    return _classifier(s1_parts, cls_W[0:1], cls_W[1:], cls_b.reshape(1, -1))
